# Initial kernel scaffold; baseline (speedup 1.0000x reference)
import jax
import jax.numpy as jnp
from jax.experimental import pallas as pl


def kernel(inputs, tables):
    raise NotImplementedError("write your pallas kernel here")


# speedup vs baseline: 2.3844x; 2.3844x over previous
import functools

import jax
import jax.numpy as jnp
from jax import lax
from jax.experimental import pallas as pl
from jax.experimental.pallas import tpu as pltpu
from jax.experimental.pallas import tpu_sc as plsc

B, S, F, V, D = 4096, 20, 26, 100000, 32
N = B * S * F
L = 16
NC, NS = 2, 16
NW = NC * NS
ROWS_PER_W = N // NW
C = 1040
CHUNKS = ROWS_PER_W // C

@functools.cache
def _build():
    mesh = plsc.VectorSubcoreMesh(core_axis_name="c", subcore_axis_name="s")

    @functools.partial(
        pl.kernel,
        mesh=mesh,
        out_type=jax.ShapeDtypeStruct((N, D), jnp.float32),
        scratch_types=[
            pltpu.VMEM((C,), jnp.int32),
            pltpu.VMEM((C, D), jnp.float32),
            pltpu.SemaphoreType.DMA,
        ],
        compiler_params=pltpu.CompilerParams(use_tc_tiling_on_sc=False),
    )
    def _gather_kernel(idx_hbm, tab_hbm, out_hbm, idx_v, rows_v, sem):
        wid = lax.axis_index("s") * NC + lax.axis_index("c")
        wbase = wid * ROWS_PER_W

        def chunk_body(g, _):
            base = wbase + g * C
            pltpu.sync_copy(idx_hbm.at[pl.ds(base, C)], idx_v)

            def vec_body(j, _):
                pos = base + j * L + lax.iota(jnp.int32, L)
                f = lax.rem(pos, F)
                idx_v[pl.ds(j * L, L)] = idx_v[pl.ds(j * L, L)] + f * V
                return 0

            lax.fori_loop(0, C // L, vec_body, 0, unroll=False)
            pltpu.async_copy(tab_hbm.at[idx_v], rows_v, sem).wait()
            pltpu.sync_copy(rows_v, out_hbm.at[pl.ds(base, C)])
            return 0

        lax.fori_loop(0, CHUNKS, chunk_body, 0, unroll=False)

    return _gather_kernel


def kernel(inputs, tables):
    idx_flat = inputs.reshape(N)
    tab_flat = tables.reshape(F * V, D)
    out = _build()(idx_flat, tab_flat)
    return out.reshape(B, S, F, D)


# speedup vs baseline: 2.4697x; 1.0358x over previous
import functools

import jax
import jax.numpy as jnp
from jax import lax
from jax.experimental import pallas as pl
from jax.experimental.pallas import tpu as pltpu
from jax.experimental.pallas import tpu_sc as plsc

B, S, F, V, D = 4096, 20, 26, 100000, 32
N = B * S * F
L = 16
NC, NS = 2, 16
NW = NC * NS
ROWS_PER_W = N // NW
C = 1664
CHUNKS = ROWS_PER_W // C
NBUF = 2

@functools.cache
def _build():
    mesh = plsc.VectorSubcoreMesh(core_axis_name="c", subcore_axis_name="s")

    @functools.partial(
        pl.kernel,
        mesh=mesh,
        out_type=jax.ShapeDtypeStruct((N, D), jnp.float32),
        scratch_types=[
            pltpu.VMEM((C,), jnp.int32),
            pltpu.VMEM((C,), jnp.int32),
            pltpu.VMEM((C, D), jnp.float32),
            pltpu.VMEM((C, D), jnp.float32),
            pltpu.VMEM((C,), jnp.int32),
            pltpu.SemaphoreType.DMA,
            pltpu.SemaphoreType.DMA,
            pltpu.SemaphoreType.DMA,
            pltpu.SemaphoreType.DMA,
        ],
        compiler_params=pltpu.CompilerParams(use_tc_tiling_on_sc=False),
    )
    def _gather_kernel(idx_hbm, tab_hbm, out_hbm, i0, i1, r0, r1, pat,
                       gs0, gs1, os0, os1):
        idxv, rowsv = [i0, i1], [r0, r1]
        gsem, osem = [gs0, gs1], [os0, os1]
        wid = lax.axis_index("s") * NC + lax.axis_index("c")
        wbase = wid * ROWS_PER_W

        def pat_body(j, _):
            pos = j * L + lax.iota(jnp.int32, L)
            pat[pl.ds(j * L, L)] = lax.rem(pos, F) * V
            return 0

        lax.fori_loop(0, C // L, pat_body, 0, unroll=False)

        def prep_idx(g, b):
            base = wbase + g * C
            pltpu.sync_copy(idx_hbm.at[pl.ds(base, C)], idxv[b])

            def add_body(j, _):
                sl = pl.ds(j * L, L)
                idxv[b][sl] = idxv[b][sl] + pat[sl]
                return 0

            lax.fori_loop(0, C // L, add_body, 0, unroll=False)

        def issue_gather(b):
            pltpu.async_copy(tab_hbm.at[idxv[b]], rowsv[b], gsem[b])

        def finish(g, b):
            pltpu.make_async_copy(tab_hbm.at[idxv[b]], rowsv[b], gsem[b]).wait()
            base = wbase + g * C
            pltpu.async_copy(rowsv[b], out_hbm.at[pl.ds(base, C)], osem[b])

        def wait_out(g, b):
            base = wbase + g * C
            pltpu.make_async_copy(
                rowsv[b], out_hbm.at[pl.ds(base, C)], osem[b]).wait()

        for b in range(NBUF):
            prep_idx(b, b)
            issue_gather(b)

        def loop_body(g0, _):
            for b in range(NBUF):
                g = g0 * NBUF + b
                finish(g, b)
                prep_idx(g + NBUF, b)
                wait_out(g, b)
                issue_gather(b)
            return 0

        lax.fori_loop(0, CHUNKS // NBUF - 1, loop_body, 0, unroll=False)

        for b in range(NBUF):
            g = CHUNKS - NBUF + b
            finish(g, b)
            wait_out(g, b)

    return _gather_kernel


def kernel(inputs, tables):
    idx_flat = inputs.reshape(N)
    tab_flat = tables.reshape(F * V, D)
    out = _build()(idx_flat, tab_flat)
    return out.reshape(B, S, F, D)
